# baseline (speedup 1.0000x reference)
import functools

import jax
import jax.numpy as jnp
from jax import lax
from jax.experimental import pallas as pl
from jax.experimental.pallas import tpu as pltpu
from jax.experimental.pallas import tpu_sc as plsc

_BT = 256


def _sc_gather(table, idx3):
    nw, nchunks, chunk = idx3.shape
    n = nw * nchunks * chunk
    d = table.shape[1]
    info = plsc.get_sparse_core_info()
    assert nw == info.num_cores * info.num_subcores
    mesh = plsc.VectorSubcoreMesh(core_axis_name="c", subcore_axis_name="s")

    @functools.partial(
        pl.kernel,
        mesh=mesh,
        out_type=jax.ShapeDtypeStruct((n, d), table.dtype),
        scratch_types=[
            pltpu.VMEM((nchunks, chunk), jnp.int32),
            pltpu.VMEM((chunk, d), table.dtype),
            pltpu.VMEM((chunk, d), table.dtype),
            pltpu.SemaphoreType.DMA,
            pltpu.SemaphoreType.DMA,
            pltpu.SemaphoreType.DMA,
            pltpu.SemaphoreType.DMA,
        ],
    )
    def k(table_hbm, idx_hbm, out_hbm, idx_v, buf0, buf1, g0, g1, o0, o1):
        wid = lax.axis_index("s") * info.num_cores + lax.axis_index("c")
        base = wid * (nchunks * chunk)
        bufs, gsems, osems = [buf0, buf1], [g0, g1], [o0, o1]
        pltpu.sync_copy(idx_hbm.at[wid], idx_v)
        g = [None, None]
        o = [None, None]
        g[0] = pltpu.async_copy(table_hbm.at[idx_v.at[0]], bufs[0], gsems[0])
        for c in range(nchunks):
            b = c % 2
            nb_ = (c + 1) % 2
            if c + 1 < nchunks:
                if o[nb_] is not None:
                    o[nb_].wait()
                g[nb_] = pltpu.async_copy(
                    table_hbm.at[idx_v.at[c + 1]], bufs[nb_], gsems[nb_])
            g[b].wait()
            o[b] = pltpu.async_copy(
                bufs[b], out_hbm.at[pl.ds(base + c * chunk, chunk)], osems[b])
        for c in range(max(0, nchunks - 2), nchunks):
            o[c % 2].wait()

    return k(table, idx3)


def _sc_dispatch(hidden, w16, inv3, tp):
    nw, nchunks, chunk = inv3.shape
    t, d = hidden.shape
    info = plsc.get_sparse_core_info()
    mesh = plsc.VectorSubcoreMesh(core_axis_name="c", subcore_axis_name="s")

    @functools.partial(
        pl.kernel,
        mesh=mesh,
        out_type=(jax.ShapeDtypeStruct((tp, d), hidden.dtype),
                  jax.ShapeDtypeStruct((tp, 128), jnp.float32)),
        scratch_types=[
            pltpu.VMEM((nchunks, chunk), jnp.int32),
            pltpu.VMEM((chunk, d), hidden.dtype),
            pltpu.VMEM((chunk, 128), jnp.float32),
            pltpu.SemaphoreType.DMA,
            pltpu.SemaphoreType.DMA,
            pltpu.SemaphoreType.DMA,
        ],
    )
    def k(hid_hbm, w_hbm, inv_hbm, xp_hbm, wp_hbm, idx_v, rows_v, w_v,
          s0, s1, s2):
        wid = lax.axis_index("s") * info.num_cores + lax.axis_index("c")
        base = wid * (nchunks * chunk)
        pltpu.sync_copy(inv_hbm.at[wid], idx_v)
        for c in range(nchunks):
            h0 = pltpu.async_copy(
                hid_hbm.at[pl.ds(base + c * chunk, chunk)], rows_v, s0)
            h1 = pltpu.async_copy(
                w_hbm.at[pl.ds(base + c * chunk, chunk)], w_v, s1)
            h0.wait()
            h2 = pltpu.async_copy(rows_v, xp_hbm.at[idx_v.at[c]], s2)
            h1.wait()
            h3 = pltpu.async_copy(w_v, wp_hbm.at[idx_v.at[c]], s1)
            h2.wait()
            h3.wait()

    return k(hidden, w16, inv3)


def _mlp_block_kernel(be_ref, x_ref, w_ref, wg0_ref, wg1_ref, wu0_ref,
                      wu1_ref, wd0_ref, wd1_ref, o_ref):
    x = x_ref[...].astype(jnp.bfloat16)

    def dotT(a, b):
        return lax.dot_general(a, b, (((1,), (1,)), ((), ())),
                               preferred_element_type=jnp.float32)

    y = None
    for wg_ref, wu_ref, wd_ref in ((wg0_ref, wu0_ref, wd0_ref),
                                   (wg1_ref, wu1_ref, wd1_ref)):
        g = dotT(x, wg_ref[0].astype(jnp.bfloat16))
        u = dotT(x, wu_ref[0].astype(jnp.bfloat16))
        h = (g * lax.logistic(g) * u).astype(jnp.bfloat16)
        yk = dotT(h, wd_ref[0].astype(jnp.bfloat16))
        y = yk if y is None else y + yk
    o_ref[...] = y * w_ref[:, :1]


def _grouped_mlp(x_p, w_p, be, W_gate, W_up, W_down):
    tp, d = x_p.shape
    e, f, _ = W_gate.shape
    nblk = tp // _BT
    grid_spec = pltpu.PrefetchScalarGridSpec(
        num_scalar_prefetch=1,
        grid=(nblk,),
        in_specs=[
            pl.BlockSpec((_BT, d), lambda i, be: (i, 0)),
            pl.BlockSpec((_BT, 128), lambda i, be: (i, 0)),
            pl.BlockSpec((1, f // 2, d), lambda i, be: (be[i], 0, 0)),
            pl.BlockSpec((1, f // 2, d), lambda i, be: (be[i], 1, 0)),
            pl.BlockSpec((1, f // 2, d), lambda i, be: (be[i], 0, 0)),
            pl.BlockSpec((1, f // 2, d), lambda i, be: (be[i], 1, 0)),
            pl.BlockSpec((1, d, f // 2), lambda i, be: (be[i], 0, 0)),
            pl.BlockSpec((1, d, f // 2), lambda i, be: (be[i], 0, 1)),
        ],
        out_specs=pl.BlockSpec((_BT, d), lambda i, be: (i, 0)),
    )
    return pl.pallas_call(
        _mlp_block_kernel,
        grid_spec=grid_spec,
        out_shape=jax.ShapeDtypeStruct((tp, d), jnp.float32),
        compiler_params=pltpu.CompilerParams(
            dimension_semantics=("arbitrary",)),
    )(be, x_p, w_p, W_gate, W_gate, W_up, W_up, W_down, W_down)


def kernel(hidden_states, top_k_index, top_k_weights, W_gate, W_up, W_down):
    t, d = hidden_states.shape
    e = W_gate.shape[0]
    nblk = t // _BT + e
    tp = nblk * _BT

    eid = top_k_index[:, 0].astype(jnp.int32)
    onehot = (eid[:, None] == jnp.arange(e, dtype=jnp.int32)[None, :]
              ).astype(jnp.bfloat16)
    tri = (jnp.arange(t, dtype=jnp.int32)[:, None]
           >= jnp.arange(t, dtype=jnp.int32)[None, :]).astype(jnp.bfloat16)
    csum = jax.lax.dot(tri, onehot, preferred_element_type=jnp.float32)
    counts = csum[-1].astype(jnp.int32)
    rank = jnp.sum(onehot.astype(jnp.float32) * csum,
                   axis=1).astype(jnp.int32) - 1
    nb = (counts + _BT - 1) // _BT
    bstart = jnp.concatenate(
        [jnp.zeros((1,), jnp.int32), jnp.cumsum(nb).astype(jnp.int32)])
    be = jnp.sum(jnp.arange(nblk, dtype=jnp.int32)[:, None]
                 >= bstart[None, 1:], axis=1, dtype=jnp.int32)
    be = jnp.minimum(be, e - 1)
    inv_p = bstart[eid] * _BT + rank
    w16 = jnp.broadcast_to(
        top_k_weights[:, :1].astype(jnp.float32), (t, 128))

    info = plsc.get_sparse_core_info()
    nw = info.num_cores * info.num_subcores
    inv3 = inv_p.reshape(nw, -1, t // nw if t // nw <= 128 else 64)

    x_p, w_p = _sc_dispatch(hidden_states, w16, inv3, tp)
    out_p = _grouped_mlp(x_p, w_p, be, W_gate, W_up, W_down)
    out = _sc_gather(out_p, inv_p.reshape(nw, -1, 32))
    return out


# speedup vs baseline: 1.0057x; 1.0057x over previous
import functools

import jax
import jax.numpy as jnp
from jax import lax
from jax.experimental import pallas as pl
from jax.experimental.pallas import tpu as pltpu
from jax.experimental.pallas import tpu_sc as plsc

_BT = 256


def _sc_gather(table, idx3):
    nw, nchunks, chunk = idx3.shape
    n = nw * nchunks * chunk
    d = table.shape[1]
    info = plsc.get_sparse_core_info()
    assert nw == info.num_cores * info.num_subcores
    mesh = plsc.VectorSubcoreMesh(core_axis_name="c", subcore_axis_name="s")

    @functools.partial(
        pl.kernel,
        mesh=mesh,
        out_type=jax.ShapeDtypeStruct((n, d), table.dtype),
        scratch_types=[
            pltpu.VMEM((nchunks, chunk), jnp.int32),
            pltpu.VMEM((chunk, d), table.dtype),
            pltpu.VMEM((chunk, d), table.dtype),
            pltpu.SemaphoreType.DMA,
            pltpu.SemaphoreType.DMA,
            pltpu.SemaphoreType.DMA,
            pltpu.SemaphoreType.DMA,
        ],
    )
    def k(table_hbm, idx_hbm, out_hbm, idx_v, buf0, buf1, g0, g1, o0, o1):
        wid = lax.axis_index("s") * info.num_cores + lax.axis_index("c")
        base = wid * (nchunks * chunk)
        bufs, gsems, osems = [buf0, buf1], [g0, g1], [o0, o1]
        pltpu.sync_copy(idx_hbm.at[wid], idx_v)
        g = [None, None]
        o = [None, None]
        g[0] = pltpu.async_copy(table_hbm.at[idx_v.at[0]], bufs[0], gsems[0])
        for c in range(nchunks):
            b = c % 2
            nb_ = (c + 1) % 2
            if c + 1 < nchunks:
                if o[nb_] is not None:
                    o[nb_].wait()
                g[nb_] = pltpu.async_copy(
                    table_hbm.at[idx_v.at[c + 1]], bufs[nb_], gsems[nb_])
            g[b].wait()
            o[b] = pltpu.async_copy(
                bufs[b], out_hbm.at[pl.ds(base + c * chunk, chunk)], osems[b])
        for c in range(max(0, nchunks - 2), nchunks):
            o[c % 2].wait()

    return k(table, idx3)


def _sc_dispatch(hidden, w16, inv3, tp):
    nw, nchunks, chunk = inv3.shape
    t, d = hidden.shape
    info = plsc.get_sparse_core_info()
    mesh = plsc.VectorSubcoreMesh(core_axis_name="c", subcore_axis_name="s")

    @functools.partial(
        pl.kernel,
        mesh=mesh,
        out_type=(jax.ShapeDtypeStruct((tp, d), hidden.dtype),
                  jax.ShapeDtypeStruct((tp, 128), jnp.float32)),
        scratch_types=[
            pltpu.VMEM((nchunks, chunk), jnp.int32),
            pltpu.VMEM((chunk, d), hidden.dtype),
            pltpu.VMEM((chunk, 128), jnp.float32),
            pltpu.SemaphoreType.DMA,
            pltpu.SemaphoreType.DMA,
            pltpu.SemaphoreType.DMA,
        ],
    )
    def k(hid_hbm, w_hbm, inv_hbm, xp_hbm, wp_hbm, idx_v, rows_v, w_v,
          s0, s1, s2):
        wid = lax.axis_index("s") * info.num_cores + lax.axis_index("c")
        base = wid * (nchunks * chunk)
        pltpu.sync_copy(inv_hbm.at[wid], idx_v)
        for c in range(nchunks):
            h0 = pltpu.async_copy(
                hid_hbm.at[pl.ds(base + c * chunk, chunk)], rows_v, s0)
            h1 = pltpu.async_copy(
                w_hbm.at[pl.ds(base + c * chunk, chunk)], w_v, s1)
            h0.wait()
            h2 = pltpu.async_copy(rows_v, xp_hbm.at[idx_v.at[c]], s2)
            h1.wait()
            h3 = pltpu.async_copy(w_v, wp_hbm.at[idx_v.at[c]], s1)
            h2.wait()
            h3.wait()

    return k(hidden, w16, inv3)


def _mlp_block_kernel(be_ref, x_ref, w_ref, wg_ref, wu_ref, wd_ref, o_ref):
    x = x_ref[...].astype(jnp.bfloat16)

    def dotT(a, b):
        return lax.dot_general(a, b, (((1,), (1,)), ((), ())),
                               preferred_element_type=jnp.float32)

    g = dotT(x, wg_ref[0].astype(jnp.bfloat16))
    u = dotT(x, wu_ref[0].astype(jnp.bfloat16))
    h = (g * lax.logistic(g) * u).astype(jnp.bfloat16)
    y = dotT(h, wd_ref[0].astype(jnp.bfloat16))
    o_ref[...] = y * w_ref[:, :1]


def _grouped_mlp(x_p, w_p, be, W_gate, W_up, W_down):
    tp, d = x_p.shape
    e, f, _ = W_gate.shape
    nblk = tp // _BT
    grid_spec = pltpu.PrefetchScalarGridSpec(
        num_scalar_prefetch=1,
        grid=(nblk,),
        in_specs=[
            pl.BlockSpec((_BT, d), lambda i, be: (i, 0)),
            pl.BlockSpec((_BT, 128), lambda i, be: (i, 0)),
            pl.BlockSpec((1, f, d), lambda i, be: (be[i], 0, 0)),
            pl.BlockSpec((1, f, d), lambda i, be: (be[i], 0, 0)),
            pl.BlockSpec((1, d, f), lambda i, be: (be[i], 0, 0)),
        ],
        out_specs=pl.BlockSpec((_BT, d), lambda i, be: (i, 0)),
    )
    return pl.pallas_call(
        _mlp_block_kernel,
        grid_spec=grid_spec,
        out_shape=jax.ShapeDtypeStruct((tp, d), jnp.float32),
        compiler_params=pltpu.CompilerParams(
            dimension_semantics=("arbitrary",)),
    )(be, x_p, w_p, W_gate, W_up, W_down)


def kernel(hidden_states, top_k_index, top_k_weights, W_gate, W_up, W_down):
    t, d = hidden_states.shape
    e = W_gate.shape[0]
    nblk = t // _BT + e
    tp = nblk * _BT

    eid = top_k_index[:, 0].astype(jnp.int32)
    onehot = (eid[:, None] == jnp.arange(e, dtype=jnp.int32)[None, :]
              ).astype(jnp.bfloat16)
    tri = (jnp.arange(t, dtype=jnp.int32)[:, None]
           >= jnp.arange(t, dtype=jnp.int32)[None, :]).astype(jnp.bfloat16)
    csum = jax.lax.dot(tri, onehot, preferred_element_type=jnp.float32)
    counts = csum[-1].astype(jnp.int32)
    rank = jnp.sum(onehot.astype(jnp.float32) * csum,
                   axis=1).astype(jnp.int32) - 1
    nb = (counts + _BT - 1) // _BT
    bstart = jnp.concatenate(
        [jnp.zeros((1,), jnp.int32), jnp.cumsum(nb).astype(jnp.int32)])
    be = jnp.sum(jnp.arange(nblk, dtype=jnp.int32)[:, None]
                 >= bstart[None, 1:], axis=1, dtype=jnp.int32)
    be = jnp.minimum(be, e - 1)
    inv_p = bstart[eid] * _BT + rank
    w16 = jnp.broadcast_to(
        top_k_weights[:, :1].astype(jnp.float32), (t, 128))

    info = plsc.get_sparse_core_info()
    nw = info.num_cores * info.num_subcores
    inv3 = inv_p.reshape(nw, -1, t // nw if t // nw <= 128 else 64)

    x_p, w_p = _sc_dispatch(hidden_states, w16, inv3, tp)
    out_p = _grouped_mlp(x_p, w_p, be, W_gate, W_up, W_down)
    out = _sc_gather(out_p, inv_p.reshape(nw, -1, 32))
    return out
